# Initial kernel scaffold; baseline (speedup 1.0000x reference)
import jax
import jax.numpy as jnp
from jax.experimental import pallas as pl


def kernel(x, edge_index, W1, b1, W2, b2, Wl, bl):
    raise NotImplementedError("write your pallas kernel here")


# speedup vs baseline: 11.5727x; 11.5727x over previous
import dataclasses
import functools

import jax
import jax.numpy as jnp
from jax import lax
from jax.experimental import pallas as pl
from jax.experimental.pallas import tpu as pltpu
from jax.experimental.pallas import tpu_sc as plsc

N = 50000
NP = 50048
E = 1600000
F_IN = 37
F1P = 128
HID = 128
OUT = 2

NC = 2
NS = 16
NCHUNK = 4
NPC = NP // NCHUNK
DR = 784
DR_LAST = NPC - 15 * DR

K = 128
EB = 400
E_PER_TILE = E // NS
NB = E_PER_TILE // EB
CAP = 5 * K + 16

TN = 1000
GRID = N // TN

_sc_mesh = plsc.VectorSubcoreMesh(core_axis_name="c", subcore_axis_name="s")
_SC_PARAMS = dataclasses.replace(pltpu.CompilerParams(), needs_layout_passes=False)


def _make_sc_pass(wz, with_gather):
    scratch = []
    if with_gather:
        scratch.append(pltpu.VMEM((EB,), jnp.int32))
    scratch += [
        pltpu.VMEM((EB,), jnp.int32),
        pltpu.VMEM((CAP,), jnp.int32),
        pltpu.VMEM((CAP,), jnp.int32),
        pltpu.VMEM((K,), jnp.int32),
        pltpu.VMEM((K, wz), jnp.float32),
        pltpu.VMEM_SHARED((NPC + 8, wz), jnp.float32),
        pltpu.SemaphoreType.DMA,
    ]

    @functools.partial(
        pl.kernel,
        out_type=jax.ShapeDtypeStruct((NP, wz), jnp.float32),
        mesh=_sc_mesh,
        scratch_types=scratch,
        compiler_params=_SC_PARAMS,
    )
    def sc_pass(*refs):
        if with_gather:
            (z_hbm, src_hbm, dst_hbm, zeros_hbm, out_hbm,
             src_v, dst_v, csrc, cdst, cfire, rows_v, acc, sem) = refs
        else:
            (const_hbm, dst_hbm, zeros_hbm, out_hbm,
             dst_v, csrc, cdst, cfire, rows_v, acc, sem) = refs

        c = lax.axis_index("c")
        s = lax.axis_index("s")
        ebase = s * E_PER_TILE

        if not with_gather:
            pltpu.sync_copy(const_hbm, rows_v)

        def fire(off):
            if with_gather:
                pltpu.async_copy(z_hbm.at[csrc.at[pl.ds(off, K)]],
                                 rows_v, sem).wait()

            @pl.loop(0, K, step=16)
            def _(i):
                cfire[pl.ds(i, 16)] = cdst[pl.ds(off + i, 16)]

            pltpu.sync_copy(rows_v, acc.at[cfire], add=True)

        for jj in range(NCHUNK // NC):
            q = c * (NCHUNK // NC) + jj
            lo = q * NPC

            @pl.when(s < NS - 1)
            def _():
                pltpu.sync_copy(zeros_hbm.at[pl.ds(0, DR)],
                                acc.at[pl.ds(s * DR, DR)])

            @pl.when(s == NS - 1)
            def _():
                pltpu.sync_copy(zeros_hbm.at[pl.ds(0, DR_LAST)],
                                acc.at[pl.ds(15 * DR, DR_LAST)])

            plsc.subcore_barrier()

            def block_body(b, p):
                pltpu.sync_copy(dst_hbm.at[pl.ds(ebase + b * EB, EB)], dst_v)
                if with_gather:
                    pltpu.sync_copy(src_hbm.at[pl.ds(ebase + b * EB, EB)],
                                    src_v)

                def grp(g, p):
                    d16 = dst_v[pl.ds(g * 16, 16)]
                    m = (d16 >= lo) & (d16 < lo + NPC)
                    plsc.store_compressed(cdst.at[pl.ds(p, 16)], d16 - lo, mask=m)
                    if with_gather:
                        s16 = src_v[pl.ds(g * 16, 16)]
                        plsc.store_compressed(csrc.at[pl.ds(p, 16)], s16, mask=m)
                    return p + jnp.sum(m.astype(jnp.int32))

                p = lax.fori_loop(0, EB // 16, grp, p)

                for j in range(4):
                    @pl.when(p >= (j + 1) * K)
                    def _():
                        fire(j * K)

                nk = lax.shift_right_logical(p, 7)

                @pl.when(nk > 0)
                def _():
                    @pl.loop(0, K + 16, step=16)
                    def _(i):
                        cdst[pl.ds(i, 16)] = cdst[pl.ds(nk * K + i, 16)]
                        if with_gather:
                            csrc[pl.ds(i, 16)] = csrc[pl.ds(nk * K + i, 16)]

                return p - nk * K

            p = lax.fori_loop(0, NB, block_body, jnp.int32(0))

            @pl.when(p > 0)
            def _():
                def padgrp(i, _):
                    off = i * 16
                    io = lax.iota(jnp.int32, 16) + off
                    mf = io >= p
                    cdst[pl.ds(off, 16)] = jnp.where(
                        mf, jnp.int32(NPC), cdst[pl.ds(off, 16)])
                    if with_gather:
                        csrc[pl.ds(off, 16)] = jnp.where(
                            mf, jnp.int32(0), csrc[pl.ds(off, 16)])
                    return 0

                lax.fori_loop(0, K // 16, padgrp, 0)
                fire(0)

            plsc.subcore_barrier()

            @pl.when(s < NS - 1)
            def _():
                pltpu.sync_copy(acc.at[pl.ds(s * DR, DR)],
                                out_hbm.at[pl.ds(lo + s * DR, DR)])

            @pl.when(s == NS - 1)
            def _():
                pltpu.sync_copy(acc.at[pl.ds(15 * DR, DR_LAST)],
                                out_hbm.at[pl.ds(lo + 15 * DR, DR_LAST)])

            plsc.subcore_barrier()

    return sc_pass


_sc_deg = _make_sc_pass(HID, with_gather=False)
_sc_agg = _make_sc_pass(HID, with_gather=True)


def _tc_a_body(deg_ref, x_ref, dinv_ref, z1_ref):
    deg = deg_ref[...][:, 0] + 1.0
    dinv = lax.rsqrt(deg)[:, None]
    dinv_ref[...] = dinv
    z = x_ref[...] * dinv
    z1_ref[...] = jnp.concatenate(
        [z, jnp.zeros((TN, F1P - F_IN), jnp.float32)], axis=1)


def _tc_a(deg, x):
    return pl.pallas_call(
        _tc_a_body,
        grid=(GRID,),
        in_specs=[
            pl.BlockSpec((TN, HID), lambda i: (i, 0)),
            pl.BlockSpec((TN, F_IN), lambda i: (i, 0)),
        ],
        out_specs=[
            pl.BlockSpec((TN, 1), lambda i: (i, 0)),
            pl.BlockSpec((TN, F1P), lambda i: (i, 0)),
        ],
        out_shape=[
            jax.ShapeDtypeStruct((N, 1), jnp.float32),
            jax.ShapeDtypeStruct((NP, F1P), jnp.float32),
        ],
    )(deg, x)


def _tc_b_body(agg_ref, z1_ref, dinv_ref, w1_ref, b1_ref, z2_ref):
    dinv = dinv_ref[...]
    u = (agg_ref[...] + z1_ref[...]) * dinv
    h = jnp.dot(u, w1_ref[...], preferred_element_type=jnp.float32,
                precision=lax.Precision.HIGHEST)
    z2_ref[...] = jnp.maximum(h + b1_ref[...], 0.0) * dinv


def _tc_b(agg1, z1, dinv, w1p, b1):
    return pl.pallas_call(
        _tc_b_body,
        grid=(GRID,),
        in_specs=[
            pl.BlockSpec((TN, F1P), lambda i: (i, 0)),
            pl.BlockSpec((TN, F1P), lambda i: (i, 0)),
            pl.BlockSpec((TN, 1), lambda i: (i, 0)),
            pl.BlockSpec((F1P, HID), lambda i: (0, 0)),
            pl.BlockSpec((1, HID), lambda i: (0, 0)),
        ],
        out_specs=pl.BlockSpec((TN, HID), lambda i: (i, 0)),
        out_shape=jax.ShapeDtypeStruct((NP, HID), jnp.float32),
    )(agg1, z1, dinv, w1p, b1)


def _tc_c_body(agg_ref, z2_ref, dinv_ref, w2_ref, b2_ref, wl_ref, bl_ref,
               o_ref):
    dinv = dinv_ref[...]
    u = (agg_ref[...] + z2_ref[...]) * dinv
    h = jnp.dot(u, w2_ref[...], preferred_element_type=jnp.float32,
                precision=lax.Precision.HIGHEST)
    h = jnp.maximum(h + b2_ref[...], 0.0)
    o = jnp.dot(h, wl_ref[...], preferred_element_type=jnp.float32,
                precision=lax.Precision.HIGHEST)
    o_ref[...] = jax.nn.sigmoid(o[:, :OUT] + bl_ref[...])


def _tc_c(agg2, z2, dinv, w2, b2, wlp, bl):
    return pl.pallas_call(
        _tc_c_body,
        grid=(GRID,),
        in_specs=[
            pl.BlockSpec((TN, HID), lambda i: (i, 0)),
            pl.BlockSpec((TN, HID), lambda i: (i, 0)),
            pl.BlockSpec((TN, 1), lambda i: (i, 0)),
            pl.BlockSpec((HID, HID), lambda i: (0, 0)),
            pl.BlockSpec((1, HID), lambda i: (0, 0)),
            pl.BlockSpec((HID, HID), lambda i: (0, 0)),
            pl.BlockSpec((1, OUT), lambda i: (0, 0)),
        ],
        out_specs=pl.BlockSpec((TN, OUT), lambda i: (i, 0)),
        out_shape=jax.ShapeDtypeStruct((N, OUT), jnp.float32),
    )(agg2, z2, dinv, w2, b2, wlp, bl)


def kernel(x, edge_index, W1, b1, W2, b2, Wl, bl):
    src = edge_index[0]
    dst = edge_index[1]

    ones_rows = jnp.concatenate(
        [jnp.ones((K, 1), jnp.float32), jnp.zeros((K, HID - 1), jnp.float32)],
        axis=1)
    zeros_rows = jnp.zeros((DR, HID), jnp.float32)
    w1p = jnp.pad(W1, ((0, F1P - F_IN), (0, 0)))
    wlp = jnp.pad(Wl, ((0, 0), (0, HID - OUT)))

    deg = _sc_deg(ones_rows, dst, zeros_rows)
    dinv, z1 = _tc_a(deg, x)

    agg1 = _sc_agg(z1, src, dst, zeros_rows)
    z2 = _tc_b(agg1, z1, dinv, w1p, b1[None, :])

    agg2 = _sc_agg(z2, src, dst, zeros_rows)
    out = _tc_c(agg2, z2, dinv, W2, b2[None, :], wlp, bl[None, :])
    return out


# speedup vs baseline: 24.8009x; 2.1431x over previous
import dataclasses
import functools

import jax
import jax.numpy as jnp
from jax import lax
from jax.experimental import pallas as pl
from jax.experimental.pallas import tpu as pltpu
from jax.experimental.pallas import tpu_sc as plsc

N = 50000
NP = 50048
E = 1600000
F_IN = 37
F1P = 128
HID = 128
OUT = 2

NC = 2
NS = 16
NCHUNK = 4
NPC = NP // NCHUNK
DR = 784
DR_LAST = NPC - 15 * DR

K = 96
EB = 400
NFIRE = 5
E_PER_TILE = E // NS
NB = E_PER_TILE // EB
CAP = (NFIRE + 1) * K + 16

TN = 1000
GRID = N // TN

_sc_mesh = plsc.VectorSubcoreMesh(core_axis_name="c", subcore_axis_name="s")
_SC_PARAMS = dataclasses.replace(pltpu.CompilerParams(), needs_layout_passes=False)


def _make_sc_pass(wz, with_gather):
    scratch = []
    if with_gather:
        scratch += [pltpu.VMEM((EB,), jnp.int32)] * 2
    scratch += [pltpu.VMEM((EB,), jnp.int32)] * 2
    scratch += [
        pltpu.VMEM((CAP,), jnp.int32),
        pltpu.VMEM((CAP,), jnp.int32),
    ]
    if with_gather:
        scratch += [pltpu.VMEM((K,), jnp.int32)] * 2
    scratch += [pltpu.VMEM((K,), jnp.int32)] * 2
    if with_gather:
        scratch += [pltpu.VMEM((K, wz), jnp.float32)] * 2
    else:
        scratch += [pltpu.VMEM((K, wz), jnp.float32)]
    scratch += [pltpu.VMEM_SHARED((NPC + 8, wz), jnp.float32)]
    scratch += [pltpu.SemaphoreType.DMA] * 2
    if with_gather:
        scratch += [pltpu.SemaphoreType.DMA] * 2
    scratch += [pltpu.SemaphoreType.DMA] * 2

    @functools.partial(
        pl.kernel,
        out_type=jax.ShapeDtypeStruct((NP, wz), jnp.float32),
        mesh=_sc_mesh,
        scratch_types=scratch,
        compiler_params=_SC_PARAMS,
    )
    def sc_pass(*refs):
        if with_gather:
            (z_hbm, src_hbm, dst_hbm, zeros_hbm, out_hbm,
             src0, src1, dst0, dst1, csrc, cdst, gf0, gf1, cf0, cf1,
             rw0, rw1, acc, esem0, esem1, gsem0, gsem1, ssem0, ssem1) = refs
            srcL, gfL, rwL = [src0, src1], [gf0, gf1], [rw0, rw1]
            gsemL = [gsem0, gsem1]
        else:
            (const_hbm, dst_hbm, zeros_hbm, out_hbm,
             dst0, dst1, csrc, cdst, cf0, cf1,
             rows_const, acc, esem0, esem1, ssem0, ssem1) = refs
        dstL, cfL = [dst0, dst1], [cf0, cf1]
        esemL, ssemL = [esem0, esem1], [ssem0, ssem1]

        c = lax.axis_index("c")
        s = lax.axis_index("s")
        ebase = s * E_PER_TILE

        if not with_gather:
            pltpu.sync_copy(const_hbm, rows_const)

        def wait_gather(P):
            pltpu.make_async_copy(z_hbm.at[gfL[P]], rwL[P], gsemL[P]).wait()

        def wait_scatter(P):
            rows = rwL[P] if with_gather else rows_const
            pltpu.make_async_copy(out_hbm.at[pl.ds(0, K)], rows,
                                  ssemL[P]).wait()

        def enq_scatter(P):
            rows = rwL[P] if with_gather else rows_const
            pltpu.async_copy(rows, acc.at[cfL[P]], ssemL[P], add=True)

        def load_eblk(bidx, S):
            pltpu.async_copy(dst_hbm.at[pl.ds(ebase + bidx * EB, EB)],
                             dstL[S], esemL[S])
            if with_gather:
                pltpu.async_copy(src_hbm.at[pl.ds(ebase + bidx * EB, EB)],
                                 srcL[S], esemL[S])

        def wait_eblk(S):
            pltpu.make_async_copy(dst_hbm.at[pl.ds(0, EB)], dstL[S],
                                  esemL[S]).wait()
            if with_gather:
                pltpu.make_async_copy(src_hbm.at[pl.ds(0, EB)], srcL[S],
                                      esemL[S]).wait()

        def fire_async(off, fcv):
            par = lax.rem(fcv, 2)
            for P in (0, 1):
                @pl.when(par == P)
                def _(P=P):
                    @pl.when(fcv >= 2)
                    def _():
                        wait_scatter(P)

                    @pl.loop(0, K, step=16)
                    def _(i):
                        cfL[P][pl.ds(i, 16)] = cdst[pl.ds(off + i, 16)]
                        if with_gather:
                            gfL[P][pl.ds(i, 16)] = csrc[pl.ds(off + i, 16)]

                    if with_gather:
                        pltpu.async_copy(z_hbm.at[gfL[P]], rwL[P], gsemL[P])

                        @pl.when(fcv >= 1)
                        def _():
                            wait_gather(1 - P)
                            enq_scatter(1 - P)
                    else:
                        enq_scatter(P)

        def compact_and_fire(S, p, fc):
            def grp(g, pp):
                d16 = dstL[S][pl.ds(g * 16, 16)]
                m = (d16 >= lo) & (d16 < lo + NPC)
                plsc.store_compressed(cdst.at[pl.ds(pp, 16)], d16 - lo,
                                      mask=m)
                if with_gather:
                    s16 = srcL[S][pl.ds(g * 16, 16)]
                    plsc.store_compressed(csrc.at[pl.ds(pp, 16)], s16, mask=m)
                return pp + jnp.sum(m.astype(jnp.int32))

            p = lax.fori_loop(0, EB // 16, grp, p)

            for j in range(NFIRE):
                @pl.when(p >= (j + 1) * K)
                def _(j=j):
                    fire_async(j * K, fc + j)

            nk = lax.div(p, jnp.int32(K))
            fc = fc + nk

            @pl.when(nk > 0)
            def _():
                @pl.loop(0, K + 16, step=16)
                def _(i):
                    cdst[pl.ds(i, 16)] = cdst[pl.ds(nk * K + i, 16)]
                    if with_gather:
                        csrc[pl.ds(i, 16)] = csrc[pl.ds(nk * K + i, 16)]

            return p - nk * K, fc

        for jj in range(NCHUNK // NC):
            q = c * (NCHUNK // NC) + jj
            lo = q * NPC

            @pl.when(s < NS - 1)
            def _():
                pltpu.sync_copy(zeros_hbm.at[pl.ds(0, DR)],
                                acc.at[pl.ds(s * DR, DR)])

            @pl.when(s == NS - 1)
            def _():
                pltpu.sync_copy(zeros_hbm.at[pl.ds(0, DR_LAST)],
                                acc.at[pl.ds(15 * DR, DR_LAST)])

            plsc.subcore_barrier()

            load_eblk(0, 0)

            def b2_body(b2, carry):
                p, fc = carry
                for S in (0, 1):
                    bidx = b2 * 2 + S

                    @pl.when(bidx + 1 < NB)
                    def _(S=S, bidx=bidx):
                        load_eblk(bidx + 1, 1 - S)

                    wait_eblk(S)
                    p, fc = compact_and_fire(S, p, fc)
                return p, fc

            p, fc = lax.fori_loop(0, NB // 2, b2_body,
                                  (jnp.int32(0), jnp.int32(0)))

            @pl.when(p > 0)
            def _():
                def padgrp(i, _):
                    off = i * 16
                    io = lax.iota(jnp.int32, 16) + off
                    mf = io >= p
                    cdst[pl.ds(off, 16)] = jnp.where(
                        mf, jnp.int32(NPC), cdst[pl.ds(off, 16)])
                    if with_gather:
                        csrc[pl.ds(off, 16)] = jnp.where(
                            mf, jnp.int32(0), csrc[pl.ds(off, 16)])
                    return 0

                lax.fori_loop(0, K // 16, padgrp, 0)
                fire_async(0, fc)

            fcf = fc + (p > 0).astype(jnp.int32)

            @pl.when(fcf >= 1)
            def _():
                par1 = lax.rem(fcf - 1, 2)
                for P in (0, 1):
                    @pl.when(par1 == P)
                    def _(P=P):
                        if with_gather:
                            wait_gather(P)
                            enq_scatter(P)

                @pl.when(fcf >= 2)
                def _():
                    par2 = lax.rem(fcf, 2)
                    for P in (0, 1):
                        @pl.when(par2 == P)
                        def _(P=P):
                            wait_scatter(P)

                for P in (0, 1):
                    @pl.when(par1 == P)
                    def _(P=P):
                        wait_scatter(P)

            plsc.subcore_barrier()

            @pl.when(s < NS - 1)
            def _():
                pltpu.sync_copy(acc.at[pl.ds(s * DR, DR)],
                                out_hbm.at[pl.ds(lo + s * DR, DR)])

            @pl.when(s == NS - 1)
            def _():
                pltpu.sync_copy(acc.at[pl.ds(15 * DR, DR_LAST)],
                                out_hbm.at[pl.ds(lo + 15 * DR, DR_LAST)])

            plsc.subcore_barrier()

    return sc_pass


_sc_deg = _make_sc_pass(HID, with_gather=False)
_sc_agg = _make_sc_pass(HID, with_gather=True)


def _tc_a_body(deg_ref, x_ref, dinv_ref, z1_ref):
    deg = deg_ref[...][:, 0] + 1.0
    dinv = lax.rsqrt(deg)[:, None]
    dinv_ref[...] = dinv
    z = x_ref[...] * dinv
    z1_ref[...] = jnp.concatenate(
        [z, jnp.zeros((TN, F1P - F_IN), jnp.float32)], axis=1)


def _tc_a(deg, x):
    return pl.pallas_call(
        _tc_a_body,
        grid=(GRID,),
        in_specs=[
            pl.BlockSpec((TN, HID), lambda i: (i, 0)),
            pl.BlockSpec((TN, F_IN), lambda i: (i, 0)),
        ],
        out_specs=[
            pl.BlockSpec((TN, 1), lambda i: (i, 0)),
            pl.BlockSpec((TN, F1P), lambda i: (i, 0)),
        ],
        out_shape=[
            jax.ShapeDtypeStruct((N, 1), jnp.float32),
            jax.ShapeDtypeStruct((NP, F1P), jnp.float32),
        ],
    )(deg, x)


def _tc_b_body(agg_ref, z1_ref, dinv_ref, w1_ref, b1_ref, z2_ref):
    dinv = dinv_ref[...]
    u = (agg_ref[...] + z1_ref[...]) * dinv
    h = jnp.dot(u, w1_ref[...], preferred_element_type=jnp.float32,
                precision=lax.Precision.HIGHEST)
    z2_ref[...] = jnp.maximum(h + b1_ref[...], 0.0) * dinv


def _tc_b(agg1, z1, dinv, w1p, b1):
    return pl.pallas_call(
        _tc_b_body,
        grid=(GRID,),
        in_specs=[
            pl.BlockSpec((TN, F1P), lambda i: (i, 0)),
            pl.BlockSpec((TN, F1P), lambda i: (i, 0)),
            pl.BlockSpec((TN, 1), lambda i: (i, 0)),
            pl.BlockSpec((F1P, HID), lambda i: (0, 0)),
            pl.BlockSpec((1, HID), lambda i: (0, 0)),
        ],
        out_specs=pl.BlockSpec((TN, HID), lambda i: (i, 0)),
        out_shape=jax.ShapeDtypeStruct((NP, HID), jnp.float32),
    )(agg1, z1, dinv, w1p, b1)


def _tc_c_body(agg_ref, z2_ref, dinv_ref, w2_ref, b2_ref, wl_ref, bl_ref,
               o_ref):
    dinv = dinv_ref[...]
    u = (agg_ref[...] + z2_ref[...]) * dinv
    h = jnp.dot(u, w2_ref[...], preferred_element_type=jnp.float32,
                precision=lax.Precision.HIGHEST)
    h = jnp.maximum(h + b2_ref[...], 0.0)
    o = jnp.dot(h, wl_ref[...], preferred_element_type=jnp.float32,
                precision=lax.Precision.HIGHEST)
    o_ref[...] = jax.nn.sigmoid(o[:, :OUT] + bl_ref[...])


def _tc_c(agg2, z2, dinv, w2, b2, wlp, bl):
    return pl.pallas_call(
        _tc_c_body,
        grid=(GRID,),
        in_specs=[
            pl.BlockSpec((TN, HID), lambda i: (i, 0)),
            pl.BlockSpec((TN, HID), lambda i: (i, 0)),
            pl.BlockSpec((TN, 1), lambda i: (i, 0)),
            pl.BlockSpec((HID, HID), lambda i: (0, 0)),
            pl.BlockSpec((1, HID), lambda i: (0, 0)),
            pl.BlockSpec((HID, HID), lambda i: (0, 0)),
            pl.BlockSpec((1, OUT), lambda i: (0, 0)),
        ],
        out_specs=pl.BlockSpec((TN, OUT), lambda i: (i, 0)),
        out_shape=jax.ShapeDtypeStruct((N, OUT), jnp.float32),
    )(agg2, z2, dinv, w2, b2, wlp, bl)


def kernel(x, edge_index, W1, b1, W2, b2, Wl, bl):
    src = edge_index[0]
    dst = edge_index[1]

    ones_rows = jnp.concatenate(
        [jnp.ones((K, 1), jnp.float32), jnp.zeros((K, HID - 1), jnp.float32)],
        axis=1)
    zeros_rows = jnp.zeros((DR, HID), jnp.float32)
    w1p = jnp.pad(W1, ((0, F1P - F_IN), (0, 0)))
    wlp = jnp.pad(Wl, ((0, 0), (0, HID - OUT)))

    deg = _sc_deg(ones_rows, dst, zeros_rows)
    dinv, z1 = _tc_a(deg, x)

    agg1 = _sc_agg(z1, src, dst, zeros_rows)
    z2 = _tc_b(agg1, z1, dinv, w1p, b1[None, :])

    agg2 = _sc_agg(z2, src, dst, zeros_rows)
    out = _tc_c(agg2, z2, dinv, W2, b2[None, :], wlp, bl[None, :])
    return out


# speedup vs baseline: 29.8565x; 1.2038x over previous
import dataclasses
import functools

import jax
import jax.numpy as jnp
from jax import lax
from jax.experimental import pallas as pl
from jax.experimental.pallas import tpu as pltpu
from jax.experimental.pallas import tpu_sc as plsc

N = 50000
NP = 50048
E = 1600000
F_IN = 37
F1P = 128
HID = 128
OUT = 2

NC = 2
NS = 16
NCHUNK = 4
NPC = NP // NCHUNK
DR = 784
DR_LAST = NPC - 15 * DR

K = 96
EB = 400
NFIRE = 5
E_PER_TILE = E // NS
NB = E_PER_TILE // EB
CAP = (NFIRE + 1) * K + 16

TN = 1024
GRID = (N + TN - 1) // TN

_sc_mesh = plsc.VectorSubcoreMesh(core_axis_name="c", subcore_axis_name="s")
_SC_PARAMS = dataclasses.replace(pltpu.CompilerParams(), needs_layout_passes=False)


def _make_sc_pass(wz, with_gather):
    scratch = []
    if with_gather:
        scratch += [pltpu.VMEM((EB,), jnp.int32)] * 2
    scratch += [pltpu.VMEM((EB,), jnp.int32)] * 2
    scratch += [
        pltpu.VMEM((CAP,), jnp.int32),
        pltpu.VMEM((CAP,), jnp.int32),
    ]
    if with_gather:
        scratch += [pltpu.VMEM((K,), jnp.int32)] * 2
    scratch += [pltpu.VMEM((K,), jnp.int32)] * 2
    if with_gather:
        scratch += [pltpu.VMEM((K, wz), jnp.float32)] * 2
    else:
        scratch += [pltpu.VMEM((K, wz), jnp.float32)]
    scratch += [pltpu.VMEM_SHARED((NPC + 8, wz), jnp.float32)]
    scratch += [pltpu.SemaphoreType.DMA] * 2
    if with_gather:
        scratch += [pltpu.SemaphoreType.DMA] * 2
    scratch += [pltpu.SemaphoreType.DMA] * 2

    @functools.partial(
        pl.kernel,
        out_type=jax.ShapeDtypeStruct((NP, wz), jnp.float32),
        mesh=_sc_mesh,
        scratch_types=scratch,
        compiler_params=_SC_PARAMS,
    )
    def sc_pass(*refs):
        if with_gather:
            (z_hbm, src_hbm, dst_hbm, zeros_hbm, out_hbm,
             src0, src1, dst0, dst1, csrc, cdst, gf0, gf1, cf0, cf1,
             rw0, rw1, acc, esem0, esem1, gsem0, gsem1, ssem0, ssem1) = refs
            srcL, gfL, rwL = [src0, src1], [gf0, gf1], [rw0, rw1]
            gsemL = [gsem0, gsem1]
        else:
            (const_hbm, dst_hbm, zeros_hbm, out_hbm,
             dst0, dst1, csrc, cdst, cf0, cf1,
             rows_const, acc, esem0, esem1, ssem0, ssem1) = refs
        dstL, cfL = [dst0, dst1], [cf0, cf1]
        esemL, ssemL = [esem0, esem1], [ssem0, ssem1]

        c = lax.axis_index("c")
        s = lax.axis_index("s")
        ebase = s * E_PER_TILE

        if not with_gather:
            pltpu.sync_copy(const_hbm, rows_const)

        def wait_gather(P):
            pltpu.make_async_copy(z_hbm.at[gfL[P]], rwL[P], gsemL[P]).wait()

        def wait_scatter(P):
            rows = rwL[P] if with_gather else rows_const
            pltpu.make_async_copy(out_hbm.at[pl.ds(0, K)], rows,
                                  ssemL[P]).wait()

        def enq_scatter(P):
            rows = rwL[P] if with_gather else rows_const
            pltpu.async_copy(rows, acc.at[cfL[P]], ssemL[P], add=True)

        def load_eblk(bidx, S):
            pltpu.async_copy(dst_hbm.at[pl.ds(ebase + bidx * EB, EB)],
                             dstL[S], esemL[S])
            if with_gather:
                pltpu.async_copy(src_hbm.at[pl.ds(ebase + bidx * EB, EB)],
                                 srcL[S], esemL[S])

        def wait_eblk(S):
            pltpu.make_async_copy(dst_hbm.at[pl.ds(0, EB)], dstL[S],
                                  esemL[S]).wait()
            if with_gather:
                pltpu.make_async_copy(src_hbm.at[pl.ds(0, EB)], srcL[S],
                                      esemL[S]).wait()

        def fire_async(off, fcv):
            par = lax.rem(fcv, 2)
            for P in (0, 1):
                @pl.when(par == P)
                def _(P=P):
                    @pl.when(fcv >= 2)
                    def _():
                        wait_scatter(P)

                    @pl.loop(0, K, step=16)
                    def _(i):
                        cfL[P][pl.ds(i, 16)] = cdst[pl.ds(off + i, 16)]
                        if with_gather:
                            gfL[P][pl.ds(i, 16)] = csrc[pl.ds(off + i, 16)]

                    if with_gather:
                        pltpu.async_copy(z_hbm.at[gfL[P]], rwL[P], gsemL[P])

                        @pl.when(fcv >= 1)
                        def _():
                            wait_gather(1 - P)
                            enq_scatter(1 - P)
                    else:
                        enq_scatter(P)

        def compact_and_fire(S, p, fc):
            def grp(g, pp):
                d16 = dstL[S][pl.ds(g * 16, 16)]
                m = (d16 >= lo) & (d16 < lo + NPC)
                plsc.store_compressed(cdst.at[pl.ds(pp, 16)], d16 - lo,
                                      mask=m)
                if with_gather:
                    s16 = srcL[S][pl.ds(g * 16, 16)]
                    plsc.store_compressed(csrc.at[pl.ds(pp, 16)], s16, mask=m)
                return pp + jnp.sum(m.astype(jnp.int32))

            p = lax.fori_loop(0, EB // 16, grp, p)

            for j in range(NFIRE):
                @pl.when(p >= (j + 1) * K)
                def _(j=j):
                    fire_async(j * K, fc + j)

            nk = lax.div(p, jnp.int32(K))
            fc = fc + nk

            @pl.when(nk > 0)
            def _():
                @pl.loop(0, K + 16, step=16)
                def _(i):
                    cdst[pl.ds(i, 16)] = cdst[pl.ds(nk * K + i, 16)]
                    if with_gather:
                        csrc[pl.ds(i, 16)] = csrc[pl.ds(nk * K + i, 16)]

            return p - nk * K, fc

        for jj in range(NCHUNK // NC):
            q = c * (NCHUNK // NC) + jj
            lo = q * NPC

            @pl.when(s < NS - 1)
            def _():
                pltpu.sync_copy(zeros_hbm.at[pl.ds(0, DR)],
                                acc.at[pl.ds(s * DR, DR)])

            @pl.when(s == NS - 1)
            def _():
                pltpu.sync_copy(zeros_hbm.at[pl.ds(0, DR_LAST)],
                                acc.at[pl.ds(15 * DR, DR_LAST)])

            plsc.subcore_barrier()

            load_eblk(0, 0)

            def b2_body(b2, carry):
                p, fc = carry
                for S in (0, 1):
                    bidx = b2 * 2 + S

                    @pl.when(bidx + 1 < NB)
                    def _(S=S, bidx=bidx):
                        load_eblk(bidx + 1, 1 - S)

                    wait_eblk(S)
                    p, fc = compact_and_fire(S, p, fc)
                return p, fc

            p, fc = lax.fori_loop(0, NB // 2, b2_body,
                                  (jnp.int32(0), jnp.int32(0)))

            @pl.when(p > 0)
            def _():
                def padgrp(i, _):
                    off = i * 16
                    io = lax.iota(jnp.int32, 16) + off
                    mf = io >= p
                    cdst[pl.ds(off, 16)] = jnp.where(
                        mf, jnp.int32(NPC), cdst[pl.ds(off, 16)])
                    if with_gather:
                        csrc[pl.ds(off, 16)] = jnp.where(
                            mf, jnp.int32(0), csrc[pl.ds(off, 16)])
                    return 0

                lax.fori_loop(0, K // 16, padgrp, 0)
                fire_async(0, fc)

            fcf = fc + (p > 0).astype(jnp.int32)

            @pl.when(fcf >= 1)
            def _():
                par1 = lax.rem(fcf - 1, 2)
                for P in (0, 1):
                    @pl.when(par1 == P)
                    def _(P=P):
                        if with_gather:
                            wait_gather(P)
                            enq_scatter(P)

                @pl.when(fcf >= 2)
                def _():
                    par2 = lax.rem(fcf, 2)
                    for P in (0, 1):
                        @pl.when(par2 == P)
                        def _(P=P):
                            wait_scatter(P)

                for P in (0, 1):
                    @pl.when(par1 == P)
                    def _(P=P):
                        wait_scatter(P)

            plsc.subcore_barrier()

            @pl.when(s < NS - 1)
            def _():
                pltpu.sync_copy(acc.at[pl.ds(s * DR, DR)],
                                out_hbm.at[pl.ds(lo + s * DR, DR)])

            @pl.when(s == NS - 1)
            def _():
                pltpu.sync_copy(acc.at[pl.ds(15 * DR, DR_LAST)],
                                out_hbm.at[pl.ds(lo + 15 * DR, DR_LAST)])

            plsc.subcore_barrier()

    return sc_pass


_sc_agg = _make_sc_pass(HID, with_gather=True)

NW = NC * NS
E_PER_W = E // NW


@functools.partial(
    pl.kernel,
    out_type=jax.ShapeDtypeStruct((NW, 1, NP), jnp.int32),
    mesh=_sc_mesh,
    scratch_types=[
        pltpu.VMEM((E_PER_W,), jnp.int32),
        pltpu.VMEM((1, NP), jnp.int32),
        pltpu.SemaphoreType.DMA,
    ],
    compiler_params=_SC_PARAMS,
)
def _sc_deg(dst_hbm, out_hbm, dst_v, hist, sem):
    c = lax.axis_index("c")
    s = lax.axis_index("s")
    wid = s * NC + c
    pltpu.async_copy(dst_hbm.at[pl.ds(wid * E_PER_W, E_PER_W)], dst_v, sem)

    @pl.loop(0, NP, step=16)
    def _(i):
        hist[0, pl.ds(i, 16)] = jnp.zeros((16,), jnp.int32)

    pltpu.make_async_copy(dst_hbm.at[pl.ds(0, E_PER_W)], dst_v, sem).wait()

    @pl.loop(0, E_PER_W, step=16)
    def _(i):
        d16 = dst_v[pl.ds(i, 16)]
        cnt, lastm = plsc.scan_count(d16)
        plsc.addupdate_scatter(hist.at[0], [d16], cnt, mask=lastm)

    pltpu.sync_copy(hist, out_hbm.at[wid])


def _tc_a_body(deg_ref, x_ref, dinv_ref, z1_ref):
    deg = (jnp.sum(deg_ref[...], axis=0) + 1).astype(jnp.float32)
    dinv = lax.rsqrt(deg)[:, None]
    dinv_ref[...] = dinv
    z = x_ref[...] * dinv
    z1_ref[...] = jnp.concatenate(
        [z, jnp.zeros((TN, F1P - F_IN), jnp.float32)], axis=1)


def _tc_a(deg, x):
    return pl.pallas_call(
        _tc_a_body,
        grid=(GRID,),
        in_specs=[
            pl.BlockSpec((NW, TN), lambda i: (0, i)),
            pl.BlockSpec((TN, F_IN), lambda i: (i, 0)),
        ],
        out_specs=[
            pl.BlockSpec((TN, 1), lambda i: (i, 0)),
            pl.BlockSpec((TN, F1P), lambda i: (i, 0)),
        ],
        out_shape=[
            jax.ShapeDtypeStruct((N, 1), jnp.float32),
            jax.ShapeDtypeStruct((NP, F1P), jnp.float32),
        ],
    )(deg, x)


def _tc_b_body(agg_ref, z1_ref, dinv_ref, w1_ref, b1_ref, z2_ref):
    dinv = dinv_ref[...]
    u = (agg_ref[...] + z1_ref[...]) * dinv
    h = jnp.dot(u, w1_ref[...], preferred_element_type=jnp.float32,
                precision=lax.Precision.HIGHEST)
    z2_ref[...] = jnp.maximum(h + b1_ref[...], 0.0) * dinv


def _tc_b(agg1, z1, dinv, w1p, b1):
    return pl.pallas_call(
        _tc_b_body,
        grid=(GRID,),
        in_specs=[
            pl.BlockSpec((TN, F1P), lambda i: (i, 0)),
            pl.BlockSpec((TN, F1P), lambda i: (i, 0)),
            pl.BlockSpec((TN, 1), lambda i: (i, 0)),
            pl.BlockSpec((F1P, HID), lambda i: (0, 0)),
            pl.BlockSpec((1, HID), lambda i: (0, 0)),
        ],
        out_specs=pl.BlockSpec((TN, HID), lambda i: (i, 0)),
        out_shape=jax.ShapeDtypeStruct((NP, HID), jnp.float32),
    )(agg1, z1, dinv, w1p, b1)


def _tc_c_body(agg_ref, z2_ref, dinv_ref, w2_ref, b2_ref, wl_ref, bl_ref,
               o_ref):
    dinv = dinv_ref[...]
    u = (agg_ref[...] + z2_ref[...]) * dinv
    h = jnp.dot(u, w2_ref[...], preferred_element_type=jnp.float32,
                precision=lax.Precision.HIGHEST)
    h = jnp.maximum(h + b2_ref[...], 0.0)
    o = jnp.dot(h, wl_ref[...], preferred_element_type=jnp.float32,
                precision=lax.Precision.HIGHEST)
    o_ref[...] = jax.nn.sigmoid(o[:, :OUT] + bl_ref[...])


def _tc_c(agg2, z2, dinv, w2, b2, wlp, bl):
    return pl.pallas_call(
        _tc_c_body,
        grid=(GRID,),
        in_specs=[
            pl.BlockSpec((TN, HID), lambda i: (i, 0)),
            pl.BlockSpec((TN, HID), lambda i: (i, 0)),
            pl.BlockSpec((TN, 1), lambda i: (i, 0)),
            pl.BlockSpec((HID, HID), lambda i: (0, 0)),
            pl.BlockSpec((1, HID), lambda i: (0, 0)),
            pl.BlockSpec((HID, HID), lambda i: (0, 0)),
            pl.BlockSpec((1, OUT), lambda i: (0, 0)),
        ],
        out_specs=pl.BlockSpec((TN, OUT), lambda i: (i, 0)),
        out_shape=jax.ShapeDtypeStruct((N, OUT), jnp.float32),
    )(agg2, z2, dinv, w2, b2, wlp, bl)


def kernel(x, edge_index, W1, b1, W2, b2, Wl, bl):
    src = edge_index[0]
    dst = edge_index[1]

    zeros_rows = jnp.zeros((DR, HID), jnp.float32)
    w1p = jnp.pad(W1, ((0, F1P - F_IN), (0, 0)))
    wlp = jnp.pad(Wl, ((0, 0), (0, HID - OUT)))

    degp = _sc_deg(dst)
    dinv, z1 = _tc_a(degp.reshape(NW, NP), x)

    agg1 = _sc_agg(z1, src, dst, zeros_rows)
    z2 = _tc_b(agg1, z1, dinv, w1p, b1[None, :])

    agg2 = _sc_agg(z2, src, dst, zeros_rows)
    out = _tc_c(agg2, z2, dinv, W2, b2[None, :], wlp, bl[None, :])
    return out
